# Initial kernel scaffold; baseline (speedup 1.0000x reference)
import jax
import jax.numpy as jnp
from jax.experimental import pallas as pl


def kernel(user_emb, item_emb, edge_index, edge_values):
    raise NotImplementedError("write your pallas kernel here")


# speedup vs baseline: 11.6857x; 11.6857x over previous
import functools

import jax
import jax.numpy as jnp
from jax import lax
from jax.experimental import pallas as pl
from jax.experimental.pallas import tpu as pltpu
from jax.experimental.pallas import tpu_sc as plsc

N_USERS = 50000
N_NODES = 100000
N_PAD = 102400
EMB = 32
HALF = 16
N_EDGES = 1600000
GAMMA = 0.5

NC = 2
NS = 16
CHUNK = 384
GRP = CHUNK // 128
NBUF = 3
E_PAD = 1603584
EDGES_PER_TILE = E_PAD // NS
N_CHUNKS = EDGES_PER_TILE // CHUNK
N_TRIPLES = N_CHUNKS // NBUF
ROWS_PER_TILE = N_PAD // NS


def _sc_body(table, src2d, dst2d, vals, out, *bufs):
    (src_vs, dst_vs, vals_vs, rows_vs) = (bufs[0:3], bufs[3:6], bufs[6:9],
                                          bufs[9:12])
    acc = bufs[12]
    gsems = bufs[13:16]
    ssems = bufs[16:19]
    c = lax.axis_index("c")
    s = lax.axis_index("s")

    def zero_row(i, carry):
        rows_vs[0][i, :] = jnp.zeros((HALF,), jnp.float32)
        return carry
    lax.fori_loop(0, CHUNK, zero_row, 0, unroll=8)
    for k in range(16):
        pltpu.sync_copy(rows_vs[0],
                        acc.at[pl.ds(s * ROWS_PER_TILE + k * CHUNK, CHUNK)])
    pltpu.sync_copy(rows_vs[0].at[pl.ds(0, 256)],
                    acc.at[pl.ds(s * ROWS_PER_TILE + 16 * CHUNK, 256)])
    plsc.subcore_barrier()

    row_off = c * N_PAD

    def load_idx(b, i):
        rbase = s * (EDGES_PER_TILE // 128) + i * GRP
        pltpu.sync_copy(src2d.at[pl.ds(rbase, GRP)], src_vs[b])
        pltpu.sync_copy(dst2d.at[pl.ds(rbase, GRP)], dst_vs[b])
        pltpu.sync_copy(vals.at[pl.ds(rbase * 128, CHUNK)], vals_vs[b])

        def add_off(j, carry):
            g = j // (128 // HALF)
            k = j % (128 // HALF)
            src_vs[b][g, pl.ds(k * HALF, HALF)] = (
                src_vs[b][g, pl.ds(k * HALF, HALF)] + row_off)
            return carry
        lax.fori_loop(0, GRP * (128 // HALF), add_off, 0, unroll=8)

    def fire_gather(b):
        for j in range(GRP):
            pltpu.async_copy(table.at[src_vs[b].at[j]],
                             rows_vs[b].at[pl.ds(j * 128, 128)], gsems[b])

    def wait_gather(b):
        for j in range(GRP):
            pltpu.make_async_copy(table.at[src_vs[b].at[j]],
                                  rows_vs[b].at[pl.ds(j * 128, 128)],
                                  gsems[b]).wait()

    def fire_scatter(b):
        for j in range(GRP):
            pltpu.async_copy(rows_vs[b].at[pl.ds(j * 128, 128)],
                             acc.at[dst_vs[b].at[j]], ssems[b], add=True)

    def wait_scatter(b):
        for j in range(GRP):
            pltpu.make_async_copy(rows_vs[b].at[pl.ds(j * 128, 128)],
                                  acc.at[dst_vs[b].at[j]],
                                  ssems[b]).wait()

    def scale(b):
        def body(g, carry):
            v16 = vals_vs[b][pl.ds(g * HALF, HALF)]
            base = g * HALF
            for k in range(HALF):
                rows_vs[b][base + k, :] = rows_vs[b][base + k, :] * v16[k]
            return carry
        lax.fori_loop(0, CHUNK // HALF, body, 0, unroll=2)

    load_idx(0, 0)
    fire_gather(0)

    def triple(p, carry):
        for q in range(NBUF):
            i = p * NBUF + q
            nb = (q + 1) % NBUF

            @pl.when(jnp.logical_and(i >= 2, i + 1 < N_CHUNKS))
            def _():
                wait_scatter(nb)

            @pl.when(i + 1 < N_CHUNKS)
            def _():
                load_idx(nb, i + 1)
                fire_gather(nb)

            wait_gather(q)
            scale(q)
            fire_scatter(q)
        return carry
    lax.fori_loop(0, N_TRIPLES, triple, 0)

    wait_scatter((N_CHUNKS - 3) % NBUF)
    wait_scatter((N_CHUNKS - 2) % NBUF)
    wait_scatter((N_CHUNKS - 1) % NBUF)

    plsc.subcore_barrier()
    pltpu.sync_copy(acc.at[pl.ds(s * ROWS_PER_TILE, ROWS_PER_TILE)],
                    out.at[pl.ds(c * N_PAD + s * ROWS_PER_TILE,
                                 ROWS_PER_TILE)])


_sc_propagate = functools.partial(
    pl.kernel,
    out_type=jax.ShapeDtypeStruct((NC * N_PAD, HALF), jnp.float32),
    mesh=plsc.VectorSubcoreMesh(core_axis_name="c", subcore_axis_name="s"),
    scratch_types=(
        [pltpu.VMEM((GRP, 128), jnp.int32) for _ in range(NBUF)]
        + [pltpu.VMEM((GRP, 128), jnp.int32) for _ in range(NBUF)]
        + [pltpu.VMEM((CHUNK,), jnp.float32) for _ in range(NBUF)]
        + [pltpu.VMEM((CHUNK, HALF), jnp.float32) for _ in range(NBUF)]
        + [pltpu.VMEM_SHARED((N_PAD, HALF), jnp.float32)]
        + [pltpu.SemaphoreType.DMA for _ in range(2 * NBUF)]
    ),
    compiler_params=pltpu.CompilerParams(use_tc_tiling_on_sc=False,
                                         internal_scratch_in_bytes=0),
)(_sc_body)


import numpy as _np

_ROWS = 12800
_TC_B = 1600
_GRID = _ROWS // _TC_B

_i = _np.arange(256)
_S32 = (( _i[:, None] // 32) == (_i[None, :] // 32)).astype(_np.float32)
_j = _np.arange(128)
_S16 = ((_j[:, None] // 16) == (_j[None, :] // 16)).astype(_np.float32)
_k = _np.arange(256)[:, None]
_l = _np.arange(128)[None, :]
_Q1 = (_k == 32 * (_l // 16) + (_l % 16)).astype(_np.float32)
_Q2 = (_k == 32 * (_l // 16) + 16 + (_l % 16)).astype(_np.float32)
_W1 = _Q1.T.copy()
_W2 = _Q2.T.copy()

_PREC = jax.lax.Precision.HIGHEST


def _norm_first_body(x_ref, s32_ref, q1_ref, q2_ref, o_ref):
    x = x_ref[...]
    ss = jnp.dot(x * x, s32_ref[...], precision=_PREC)
    inv = 1.0 / (jnp.sqrt(ss) + 1e-12)
    y = x * inv
    o_ref[0] = jnp.dot(y, q1_ref[...], precision=_PREC)
    o_ref[1] = jnp.dot(y, q2_ref[...], precision=_PREC)


_norm_first = pl.pallas_call(
    _norm_first_body,
    out_shape=jax.ShapeDtypeStruct((NC, _ROWS, 128), jnp.float32),
    grid=(_GRID,),
    in_specs=[pl.BlockSpec((_TC_B, 256), lambda i: (i, 0)),
              pl.BlockSpec((256, 256), lambda i: (0, 0)),
              pl.BlockSpec((256, 128), lambda i: (0, 0)),
              pl.BlockSpec((256, 128), lambda i: (0, 0))],
    out_specs=pl.BlockSpec((NC, _TC_B, 128), lambda i: (0, i, 0)),
)


def _norm_mid_body(x_ref, s16_ref, o_ref):
    lo = x_ref[0]
    hi = x_ref[1]
    ss = jnp.dot(lo * lo + hi * hi, s16_ref[...], precision=_PREC)
    inv = 1.0 / (jnp.sqrt(ss) + 1e-12)
    o_ref[0] = lo * inv
    o_ref[1] = hi * inv


_norm_mid = pl.pallas_call(
    _norm_mid_body,
    out_shape=jax.ShapeDtypeStruct((NC, _ROWS, 128), jnp.float32),
    grid=(_GRID,),
    in_specs=[pl.BlockSpec((NC, _TC_B, 128), lambda i: (0, i, 0)),
              pl.BlockSpec((128, 128), lambda i: (0, 0))],
    out_specs=pl.BlockSpec((NC, _TC_B, 128), lambda i: (0, i, 0)),
)


def _combine_body(x0_ref, e1_ref, e2_ref, e3_ref, w1_ref, w2_ref, o_ref):
    plo = e1_ref[0] + e2_ref[0] + e3_ref[0]
    phi = e1_ref[1] + e2_ref[1] + e3_ref[1]
    prop = (jnp.dot(plo, w1_ref[...], precision=_PREC)
            + jnp.dot(phi, w2_ref[...], precision=_PREC))
    o_ref[...] = GAMMA * x0_ref[...] + ((1.0 - GAMMA) / 3.0) * prop


_split_spec = pl.BlockSpec((NC, _TC_B, 128), lambda i: (0, i, 0))
_combine = pl.pallas_call(
    _combine_body,
    out_shape=jax.ShapeDtypeStruct((_ROWS, 256), jnp.float32),
    grid=(_GRID,),
    in_specs=[pl.BlockSpec((_TC_B, 256), lambda i: (i, 0)),
              _split_spec, _split_spec, _split_spec,
              pl.BlockSpec((128, 256), lambda i: (0, 0)),
              pl.BlockSpec((128, 256), lambda i: (0, 0))],
    out_specs=pl.BlockSpec((_TC_B, 256), lambda i: (i, 0)),
)


def kernel(user_emb, item_emb, edge_index, edge_values):
    ego0 = jnp.concatenate([
        user_emb, item_emb,
        jnp.zeros((N_PAD - N_NODES, EMB), jnp.float32)], axis=0)
    ego0r = ego0.reshape(_ROWS, 256)
    src = edge_index[0].astype(jnp.int32)
    dst = edge_index[1].astype(jnp.int32)
    pad = E_PAD - N_EDGES
    src2d = jnp.concatenate([src, jnp.zeros((pad,), jnp.int32)]).reshape(-1, 128)
    dst2d = jnp.concatenate([dst, jnp.zeros((pad,), jnp.int32)]).reshape(-1, 128)
    vals = jnp.concatenate(
        [edge_values, jnp.zeros((pad,), jnp.float32)])

    s32 = jnp.asarray(_S32)
    s16 = jnp.asarray(_S16)
    q1 = jnp.asarray(_Q1)
    q2 = jnp.asarray(_Q2)
    w1 = jnp.asarray(_W1)
    w2 = jnp.asarray(_W2)

    t1 = _norm_first(ego0r, s32, q1, q2).reshape(NC * N_PAD, HALF)
    e1 = _sc_propagate(t1, src2d, dst2d, vals).reshape(NC, _ROWS, 128)
    t2 = _norm_mid(e1, s16).reshape(NC * N_PAD, HALF)
    e2 = _sc_propagate(t2, src2d, dst2d, vals).reshape(NC, _ROWS, 128)
    t3 = _norm_mid(e2, s16).reshape(NC * N_PAD, HALF)
    e3 = _sc_propagate(t3, src2d, dst2d, vals).reshape(NC, _ROWS, 128)

    light = _combine(ego0r, e1, e2, e3, w1, w2)
    flat = light.reshape(N_PAD, EMB)
    return (flat[:N_USERS], flat[N_USERS:N_NODES])
